# Initial kernel scaffold; baseline (speedup 1.0000x reference)
import jax
import jax.numpy as jnp
from jax.experimental import pallas as pl


def kernel(node_attributes, edge_attributes, edge_indices, edge_indices_reverse, W_e, b_e, gamma_e, beta_e, W_g1, b_g1, gamma_g1, beta_g1, W_g2, b_g2, gamma_g2, beta_g2, gamma_n, beta_n, W_nu, b_nu):
    raise NotImplementedError("write your pallas kernel here")


# speedup vs baseline: 1.7230x; 1.7230x over previous
import functools

import jax
import jax.numpy as jnp
from jax import lax
from jax.experimental import pallas as pl
from jax.experimental.pallas import tpu as pltpu
from jax.experimental.pallas import tpu_sc as plsc

EPS = 1e-3
NC = 2
NS = 16
LANES = 16


def _k1_body(x_ref, w1_ref, w2_ref, p_ref, q_ref):
    x = x_ref[...]
    p_ref[...] = jnp.dot(x, w1_ref[...], preferred_element_type=jnp.float32)
    q_ref[...] = jnp.dot(x, w2_ref[...], preferred_element_type=jnp.float32)


def _make_pq(x, w1, w2):
    n, d = x.shape
    return pl.pallas_call(
        _k1_body,
        out_shape=(
            jax.ShapeDtypeStruct((n, d), jnp.float32),
            jax.ShapeDtypeStruct((n, d), jnp.float32),
        ),
    )(x, w1, w2)


def _sc_gather_sum(p, q, idx1, idx0):
    n, d = p.shape
    e = idx1.shape[0]
    nw = NC * NS
    per = e // nw
    assert per * nw == e
    chunk = 80
    nchunks = per // chunk
    assert nchunks * chunk == per

    mesh = plsc.VectorSubcoreMesh(core_axis_name="c", subcore_axis_name="s")

    @functools.partial(
        pl.kernel,
        out_type=jax.ShapeDtypeStruct((e, d), jnp.float32),
        mesh=mesh,
        scratch_types=[
            pltpu.VMEM((chunk,), jnp.int32),
            pltpu.VMEM((chunk,), jnp.int32),
            pltpu.VMEM((chunk, d), jnp.float32),
            pltpu.VMEM((chunk, d), jnp.float32),
            pltpu.SemaphoreType.DMA,
            pltpu.SemaphoreType.DMA,
        ],
    )
    def k2(p_hbm, q_hbm, i1_hbm, i0_hbm, g_hbm, i1_v, i0_v, rp_v, rq_v, s1, s2):
        wid = lax.axis_index("s") * NC + lax.axis_index("c")
        base = wid * per

        def chunk_body(k, carry):
            off = base + k * chunk
            pltpu.sync_copy(i1_hbm.at[pl.ds(off, chunk)], i1_v)
            pltpu.sync_copy(i0_hbm.at[pl.ds(off, chunk)], i0_v)
            cp1 = pltpu.async_copy(p_hbm.at[i1_v], rp_v, s1)
            cp2 = pltpu.async_copy(q_hbm.at[i0_v], rq_v, s2)
            cp1.wait()
            cp2.wait()

            def row(r, c2):
                for j in range(d // LANES):
                    sl = pl.ds(j * LANES, LANES)
                    rp_v[r, sl] = rp_v[r, sl] + rq_v[r, sl]
                return c2

            lax.fori_loop(0, chunk, row, 0, unroll=2)
            pltpu.sync_copy(rp_v, g_hbm.at[pl.ds(off, chunk)])
            return carry

        lax.fori_loop(0, nchunks, chunk_body, 0)

    return k2(p, q, idx1, idx0)


def _k3_body(nsteps, etotal, g_ref, ea_ref, w3_ref, be_ref, gam_ref, bet_ref,
             h1_ref, s_ref, ss_ref, a_ref, c_ref):
    i = pl.program_id(0)
    h = g_ref[...] + jnp.dot(ea_ref[...], w3_ref[...],
                             preferred_element_type=jnp.float32) + be_ref[...]
    h = jnp.maximum(h, 0.0)
    h1_ref[...] = h

    @pl.when(i == 0)
    def _():
        s_ref[...] = jnp.zeros_like(s_ref)
        ss_ref[...] = jnp.zeros_like(ss_ref)

    s_ref[...] += jnp.sum(h, axis=0, keepdims=True)
    ss_ref[...] += jnp.sum(h * h, axis=0, keepdims=True)

    @pl.when(i == nsteps - 1)
    def _():
        mean = s_ref[...] / etotal
        var = ss_ref[...] / etotal - mean * mean
        a = gam_ref[...] * lax.rsqrt(var + EPS)
        a_ref[...] = a
        c_ref[...] = bet_ref[...] - mean * a


def _edge_mlp_stats(g, ea, w3, b_e, gamma_e, beta_e):
    e, d = g.shape
    de = ea.shape[1]
    blk = 2000
    nsteps = e // blk
    assert nsteps * blk == e
    body = functools.partial(_k3_body, nsteps, float(e))
    return pl.pallas_call(
        body,
        grid=(nsteps,),
        in_specs=[
            pl.BlockSpec((blk, d), lambda i: (i, 0)),
            pl.BlockSpec((blk, de), lambda i: (i, 0)),
            pl.BlockSpec((de, d), lambda i: (0, 0)),
            pl.BlockSpec((1, d), lambda i: (0, 0)),
            pl.BlockSpec((1, d), lambda i: (0, 0)),
            pl.BlockSpec((1, d), lambda i: (0, 0)),
        ],
        out_specs=[
            pl.BlockSpec((blk, d), lambda i: (i, 0)),
            pl.BlockSpec((1, d), lambda i: (0, 0)),
            pl.BlockSpec((1, d), lambda i: (0, 0)),
            pl.BlockSpec((1, d), lambda i: (0, 0)),
            pl.BlockSpec((1, d), lambda i: (0, 0)),
        ],
        out_shape=[
            jax.ShapeDtypeStruct((e, d), jnp.float32),
            jax.ShapeDtypeStruct((1, d), jnp.float32),
            jax.ShapeDtypeStruct((1, d), jnp.float32),
            jax.ShapeDtypeStruct((1, d), jnp.float32),
            jax.ShapeDtypeStruct((1, d), jnp.float32),
        ],
    )(g, ea, w3, b_e.reshape(1, d), gamma_e.reshape(1, d), beta_e.reshape(1, d))


def _sc_affine_scatter(h1, rev0, a, c, n):
    e, d = h1.shape
    nw = NC * NS
    per = e // nw
    chunk = 80
    nchunks = per // chunk
    assert nchunks * chunk == per
    zblk = 128
    rows_per_tile = ((n + NS - 1) // NS + zblk - 1) // zblk * zblk
    npad = NS * rows_per_tile
    nz = rows_per_tile // zblk

    mesh = plsc.VectorSubcoreMesh(core_axis_name="c", subcore_axis_name="s")

    @functools.partial(
        pl.kernel,
        out_type=(
            jax.ShapeDtypeStruct((e, d), jnp.float32),
            jax.ShapeDtypeStruct((NC, npad, d), jnp.float32),
        ),
        mesh=mesh,
        scratch_types=[
            pltpu.VMEM((chunk,), jnp.int32),
            pltpu.VMEM((chunk, d), jnp.float32),
            pltpu.VMEM((d,), jnp.float32),
            pltpu.VMEM((d,), jnp.float32),
            pltpu.VMEM((zblk, d), jnp.float32),
            pltpu.VMEM_SHARED((npad, d), jnp.float32),
        ],
    )
    def k4(h1_hbm, rev_hbm, a_hbm, c_hbm, eu_hbm, agg_hbm,
           idx_v, h_v, a_v, c_v, z_v, agg_sh):
        cid = lax.axis_index("c")
        sid = lax.axis_index("s")
        wid = sid * NC + cid
        base = wid * per

        pltpu.sync_copy(a_hbm, a_v)
        pltpu.sync_copy(c_hbm, c_v)

        def zrow(r, carry):
            for j in range(d // LANES):
                z_v[r, pl.ds(j * LANES, LANES)] = jnp.zeros((LANES,), jnp.float32)
            return carry

        lax.fori_loop(0, zblk, zrow, 0)
        for t in range(nz):
            pltpu.sync_copy(z_v, agg_sh.at[pl.ds(sid * rows_per_tile + t * zblk, zblk)])
        plsc.subcore_barrier()

        def chunk_body(k, carry):
            off = base + k * chunk
            pltpu.sync_copy(h1_hbm.at[pl.ds(off, chunk)], h_v)
            pltpu.sync_copy(rev_hbm.at[pl.ds(off, chunk)], idx_v)

            def row(r, c2):
                for j in range(d // LANES):
                    sl = pl.ds(j * LANES, LANES)
                    h_v[r, sl] = jnp.maximum(a_v[sl] * h_v[r, sl] + c_v[sl], 0.0)
                return c2

            lax.fori_loop(0, chunk, row, 0, unroll=2)
            pltpu.sync_copy(h_v, eu_hbm.at[pl.ds(off, chunk)])
            pltpu.sync_copy(h_v, agg_sh.at[idx_v], add=True)
            return carry

        lax.fori_loop(0, nchunks, chunk_body, 0)
        plsc.subcore_barrier()

        for t in range(nz):
            r0 = sid * rows_per_tile + t * zblk
            pltpu.sync_copy(agg_sh.at[pl.ds(r0, zblk)], z_v)
            pltpu.sync_copy(z_v, agg_hbm.at[cid, pl.ds(r0, zblk)])

    return k4(h1, rev0, a, c)


def _k5_body(x_ref, agg_ref, w1a_ref, w1b_ref, b1_ref, g1_ref, be1_ref,
             w2_ref, b2_ref, g2_ref, be2_ref, gn_ref, ben_ref,
             wn_ref, bn_ref, out_ref):
    n = x_ref.shape[0]

    def bn(x, g, b):
        m = jnp.sum(x, axis=0, keepdims=True) / n
        xc = x - m
        v = jnp.sum(xc * xc, axis=0, keepdims=True) / n
        return g * xc * lax.rsqrt(v + EPS) + b

    agg = agg_ref[0] + agg_ref[1]
    h = (jnp.dot(x_ref[...], w1a_ref[...], preferred_element_type=jnp.float32)
         + jnp.dot(agg, w1b_ref[...], preferred_element_type=jnp.float32)
         + b1_ref[...])
    h = jnp.maximum(bn(h, g1_ref[...], be1_ref[...]), 0.0)
    h = jnp.dot(h, w2_ref[...], preferred_element_type=jnp.float32) + b2_ref[...]
    h = jnp.maximum(bn(h, g2_ref[...], be2_ref[...]), 0.0)
    h = bn(h, gn_ref[...], ben_ref[...])
    h = jnp.dot(h, wn_ref[...], preferred_element_type=jnp.float32) + bn_ref[...]
    out_ref[...] = jnp.maximum(h, 0.0)


def _node_mlp(x, aggp, w1a, w1b, b1, g1, be1, w2, b2, g2, be2, gn, ben, wn, bn):
    n, d = x.shape
    r = lambda v: v.reshape(1, d)
    return pl.pallas_call(
        _k5_body,
        out_shape=jax.ShapeDtypeStruct((n, d), jnp.float32),
    )(x, aggp, w1a, w1b, r(b1), r(g1), r(be1), w2, r(b2), r(g2), r(be2),
      r(gn), r(ben), wn, r(bn))


def kernel(node_attributes, edge_attributes, edge_indices, edge_indices_reverse,
           W_e, b_e, gamma_e, beta_e, W_g1, b_g1, gamma_g1, beta_g1,
           W_g2, b_g2, gamma_g2, beta_g2, gamma_n, beta_n, W_nu, b_nu):
    n, d = node_attributes.shape
    idx1 = edge_indices[:, 1].astype(jnp.int32)
    idx0 = edge_indices[:, 0].astype(jnp.int32)
    rev0 = edge_indices_reverse[:, 0].astype(jnp.int32)

    p, q = _make_pq(node_attributes, W_e[:d], W_e[d:2 * d])
    g = _sc_gather_sum(p, q, idx1, idx0)
    h1, _, _, a, c = _edge_mlp_stats(g, edge_attributes, W_e[2 * d:],
                                     b_e, gamma_e, beta_e)
    eu, aggp = _sc_affine_scatter(h1, rev0, a.reshape(d), c.reshape(d), n)
    aggp = aggp[:, :n]
    node_final = _node_mlp(node_attributes, aggp,
                           W_g1[:d], W_g1[d:], b_g1, gamma_g1, beta_g1,
                           W_g2, b_g2, gamma_g2, beta_g2,
                           gamma_n, beta_n, W_nu, b_nu)
    return (node_final, eu)


# speedup vs baseline: 3.9769x; 2.3081x over previous
import functools

import jax
import jax.numpy as jnp
from jax import lax
from jax.experimental import pallas as pl
from jax.experimental.pallas import tpu as pltpu
from jax.experimental.pallas import tpu_sc as plsc

EPS = 1e-3
NC = 2
NS = 16
LANES = 16


def _k1_body(x_ref, w1_ref, w2_ref, p_ref, q_ref):
    x = x_ref[...]
    p_ref[...] = jnp.dot(x, w1_ref[...], preferred_element_type=jnp.float32)
    q_ref[...] = jnp.dot(x, w2_ref[...], preferred_element_type=jnp.float32)


def _make_pq(x, w1, w2):
    n, d = x.shape
    return pl.pallas_call(
        _k1_body,
        out_shape=(
            jax.ShapeDtypeStruct((n, d), jnp.float32),
            jax.ShapeDtypeStruct((n, d), jnp.float32),
        ),
    )(x, w1, w2)


def _sc_gather_sum(p, q, idx1, idx0):
    n, d = p.shape
    e = idx1.shape[0]
    nw = NC * NS
    per = e // nw
    assert per * nw == e
    chunk = 80
    nchunks = per // chunk
    assert nchunks * chunk == per
    assert nchunks % 2 == 1

    mesh = plsc.VectorSubcoreMesh(core_axis_name="c", subcore_axis_name="s")

    @functools.partial(
        pl.kernel,
        out_type=jax.ShapeDtypeStruct((e, d), jnp.float32),
        mesh=mesh,
        scratch_types=[
            pltpu.VMEM((per,), jnp.int32),
            pltpu.VMEM((per,), jnp.int32),
            pltpu.VMEM((chunk, d), jnp.float32),
            pltpu.VMEM((chunk, d), jnp.float32),
            pltpu.VMEM((chunk, d), jnp.float32),
            pltpu.VMEM((chunk, d), jnp.float32),
            pltpu.SemaphoreType.DMA,
            pltpu.SemaphoreType.DMA,
            pltpu.SemaphoreType.DMA,
            pltpu.SemaphoreType.DMA,
        ],
    )
    def k2(p_hbm, q_hbm, i1_hbm, i0_hbm, g_hbm,
           i1_v, i0_v, rp_a, rp_b, rq_a, rq_b, sp_a, sp_b, sq_a, sq_b):
        wid = lax.axis_index("s") * NC + lax.axis_index("c")
        base = wid * per

        pltpu.sync_copy(i1_hbm.at[pl.ds(base, per)], i1_v)
        pltpu.sync_copy(i0_hbm.at[pl.ds(base, per)], i0_v)

        def issue(k, rp, rq, sp, sq):
            sl = pl.ds(k * chunk, chunk)
            pltpu.async_copy(p_hbm.at[i1_v.at[sl]], rp, sp)
            pltpu.async_copy(q_hbm.at[i0_v.at[sl]], rq, sq)

        def process(k, rp, rq, sp, sq):
            sl = pl.ds(k * chunk, chunk)
            pltpu.make_async_copy(p_hbm.at[i1_v.at[sl]], rp, sp).wait()
            pltpu.make_async_copy(q_hbm.at[i0_v.at[sl]], rq, sq).wait()

            @plsc.parallel_loop(0, chunk, unroll=2)
            def _row(r):
                for j in range(d // LANES):
                    fsl = pl.ds(j * LANES, LANES)
                    rp[r, fsl] = rp[r, fsl] + rq[r, fsl]

            pltpu.sync_copy(rp, g_hbm.at[pl.ds(base + k * chunk, chunk)])

        issue(0, rp_a, rq_a, sp_a, sq_a)

        def pair(kk, carry):
            k0 = 2 * kk
            issue(k0 + 1, rp_b, rq_b, sp_b, sq_b)
            process(k0, rp_a, rq_a, sp_a, sq_a)
            issue(k0 + 2, rp_a, rq_a, sp_a, sq_a)
            process(k0 + 1, rp_b, rq_b, sp_b, sq_b)
            return carry

        lax.fori_loop(0, (nchunks - 1) // 2, pair, 0)
        process(nchunks - 1, rp_a, rq_a, sp_a, sq_a)

    return k2(p, q, idx1, idx0)


def _k3_body(nsteps, etotal, g_ref, ea_ref, w3_ref, be_ref, gam_ref, bet_ref,
             h1_ref, s_ref, ss_ref, a_ref, c_ref):
    i = pl.program_id(0)
    h = g_ref[...] + jnp.dot(ea_ref[...], w3_ref[...],
                             preferred_element_type=jnp.float32) + be_ref[...]
    h = jnp.maximum(h, 0.0)
    h1_ref[...] = h

    @pl.when(i == 0)
    def _():
        s_ref[...] = jnp.zeros_like(s_ref)
        ss_ref[...] = jnp.zeros_like(ss_ref)

    s_ref[...] += jnp.sum(h, axis=0, keepdims=True)
    ss_ref[...] += jnp.sum(h * h, axis=0, keepdims=True)

    @pl.when(i == nsteps - 1)
    def _():
        mean = s_ref[...] / etotal
        var = ss_ref[...] / etotal - mean * mean
        a = gam_ref[...] * lax.rsqrt(var + EPS)
        a_ref[...] = a
        c_ref[...] = bet_ref[...] - mean * a


def _edge_mlp_stats(g, ea, w3, b_e, gamma_e, beta_e):
    e, d = g.shape
    de = ea.shape[1]
    blk = 2000
    nsteps = e // blk
    assert nsteps * blk == e
    body = functools.partial(_k3_body, nsteps, float(e))
    return pl.pallas_call(
        body,
        grid=(nsteps,),
        in_specs=[
            pl.BlockSpec((blk, d), lambda i: (i, 0)),
            pl.BlockSpec((blk, de), lambda i: (i, 0)),
            pl.BlockSpec((de, d), lambda i: (0, 0)),
            pl.BlockSpec((1, d), lambda i: (0, 0)),
            pl.BlockSpec((1, d), lambda i: (0, 0)),
            pl.BlockSpec((1, d), lambda i: (0, 0)),
        ],
        out_specs=[
            pl.BlockSpec((blk, d), lambda i: (i, 0)),
            pl.BlockSpec((1, d), lambda i: (0, 0)),
            pl.BlockSpec((1, d), lambda i: (0, 0)),
            pl.BlockSpec((1, d), lambda i: (0, 0)),
            pl.BlockSpec((1, d), lambda i: (0, 0)),
        ],
        out_shape=[
            jax.ShapeDtypeStruct((e, d), jnp.float32),
            jax.ShapeDtypeStruct((1, d), jnp.float32),
            jax.ShapeDtypeStruct((1, d), jnp.float32),
            jax.ShapeDtypeStruct((1, d), jnp.float32),
            jax.ShapeDtypeStruct((1, d), jnp.float32),
        ],
    )(g, ea, w3, b_e.reshape(1, d), gamma_e.reshape(1, d), beta_e.reshape(1, d))


def _sc_affine_scatter(h1, rev0, a, c, n):
    e, d = h1.shape
    nw = NC * NS
    per = e // nw
    chunk = 80
    nchunks = per // chunk
    assert nchunks * chunk == per
    zblk = 128
    rows_per_tile = ((n + NS - 1) // NS + zblk - 1) // zblk * zblk
    npad = NS * rows_per_tile
    nz = rows_per_tile // zblk

    mesh = plsc.VectorSubcoreMesh(core_axis_name="c", subcore_axis_name="s")

    @functools.partial(
        pl.kernel,
        out_type=(
            jax.ShapeDtypeStruct((e, d), jnp.float32),
            jax.ShapeDtypeStruct((NC, npad, d), jnp.float32),
        ),
        mesh=mesh,
        scratch_types=[
            pltpu.VMEM((chunk,), jnp.int32),
            pltpu.VMEM((chunk,), jnp.int32),
            pltpu.VMEM((chunk, d), jnp.float32),
            pltpu.VMEM((chunk, d), jnp.float32),
            pltpu.VMEM((d,), jnp.float32),
            pltpu.VMEM((d,), jnp.float32),
            pltpu.VMEM((zblk, d), jnp.float32),
            pltpu.VMEM_SHARED((npad, d), jnp.float32),
            pltpu.SemaphoreType.DMA,
            pltpu.SemaphoreType.DMA,
            pltpu.SemaphoreType.DMA,
            pltpu.SemaphoreType.DMA,
        ],
    )
    def k4(h1_hbm, rev_hbm, a_hbm, c_hbm, eu_hbm, agg_hbm,
           idx_a, idx_b, h_a, h_b, a_v, c_v, z_v, agg_sh,
           sh_a, sh_b, si_a, si_b):
        cid = lax.axis_index("c")
        sid = lax.axis_index("s")
        wid = sid * NC + cid
        base = wid * per

        pltpu.sync_copy(a_hbm, a_v)
        pltpu.sync_copy(c_hbm, c_v)

        @plsc.parallel_loop(0, zblk, unroll=2)
        def _zrow(r):
            for j in range(d // LANES):
                z_v[r, pl.ds(j * LANES, LANES)] = jnp.zeros((LANES,), jnp.float32)

        for t in range(nz):
            pltpu.sync_copy(z_v, agg_sh.at[pl.ds(sid * rows_per_tile + t * zblk, zblk)])
        plsc.subcore_barrier()

        def issue(k, h_v, idx_v, sh, si):
            off = base + k * chunk
            pltpu.async_copy(h1_hbm.at[pl.ds(off, chunk)], h_v, sh)
            pltpu.async_copy(rev_hbm.at[pl.ds(off, chunk)], idx_v, si)

        def process(k, h_v, idx_v, sh, si):
            off = base + k * chunk
            pltpu.make_async_copy(h1_hbm.at[pl.ds(off, chunk)], h_v, sh).wait()
            pltpu.make_async_copy(rev_hbm.at[pl.ds(off, chunk)], idx_v, si).wait()

            @plsc.parallel_loop(0, chunk, unroll=2)
            def _row(r):
                for j in range(d // LANES):
                    sl = pl.ds(j * LANES, LANES)
                    h_v[r, sl] = jnp.maximum(a_v[sl] * h_v[r, sl] + c_v[sl], 0.0)

            pltpu.sync_copy(h_v, eu_hbm.at[pl.ds(off, chunk)])
            pltpu.sync_copy(h_v, agg_sh.at[idx_v], add=True)

        issue(0, h_a, idx_a, sh_a, si_a)

        def pair(kk, carry):
            k0 = 2 * kk
            issue(k0 + 1, h_b, idx_b, sh_b, si_b)
            process(k0, h_a, idx_a, sh_a, si_a)
            issue(k0 + 2, h_a, idx_a, sh_a, si_a)
            process(k0 + 1, h_b, idx_b, sh_b, si_b)
            return carry

        assert nchunks % 2 == 1
        lax.fori_loop(0, (nchunks - 1) // 2, pair, 0)
        process(nchunks - 1, h_a, idx_a, sh_a, si_a)
        plsc.subcore_barrier()

        for t in range(nz):
            r0 = sid * rows_per_tile + t * zblk
            pltpu.sync_copy(agg_sh.at[pl.ds(r0, zblk)], z_v)
            pltpu.sync_copy(z_v, agg_hbm.at[cid, pl.ds(r0, zblk)])

    return k4(h1, rev0, a, c)


def _k5_body(x_ref, agg_ref, w1a_ref, w1b_ref, b1_ref, g1_ref, be1_ref,
             w2_ref, b2_ref, g2_ref, be2_ref, gn_ref, ben_ref,
             wn_ref, bn_ref, out_ref):
    n = x_ref.shape[0]

    def bn(x, g, b):
        m = jnp.sum(x, axis=0, keepdims=True) / n
        xc = x - m
        v = jnp.sum(xc * xc, axis=0, keepdims=True) / n
        return g * xc * lax.rsqrt(v + EPS) + b

    agg = agg_ref[0] + agg_ref[1]
    h = (jnp.dot(x_ref[...], w1a_ref[...], preferred_element_type=jnp.float32)
         + jnp.dot(agg, w1b_ref[...], preferred_element_type=jnp.float32)
         + b1_ref[...])
    h = jnp.maximum(bn(h, g1_ref[...], be1_ref[...]), 0.0)
    h = jnp.dot(h, w2_ref[...], preferred_element_type=jnp.float32) + b2_ref[...]
    h = jnp.maximum(bn(h, g2_ref[...], be2_ref[...]), 0.0)
    h = bn(h, gn_ref[...], ben_ref[...])
    h = jnp.dot(h, wn_ref[...], preferred_element_type=jnp.float32) + bn_ref[...]
    out_ref[...] = jnp.maximum(h, 0.0)


def _node_mlp(x, aggp, w1a, w1b, b1, g1, be1, w2, b2, g2, be2, gn, ben, wn, bn):
    n, d = x.shape
    r = lambda v: v.reshape(1, d)
    return pl.pallas_call(
        _k5_body,
        out_shape=jax.ShapeDtypeStruct((n, d), jnp.float32),
    )(x, aggp, w1a, w1b, r(b1), r(g1), r(be1), w2, r(b2), r(g2), r(be2),
      r(gn), r(ben), wn, r(bn))


def kernel(node_attributes, edge_attributes, edge_indices, edge_indices_reverse,
           W_e, b_e, gamma_e, beta_e, W_g1, b_g1, gamma_g1, beta_g1,
           W_g2, b_g2, gamma_g2, beta_g2, gamma_n, beta_n, W_nu, b_nu):
    n, d = node_attributes.shape
    idx1 = edge_indices[:, 1].astype(jnp.int32)
    idx0 = edge_indices[:, 0].astype(jnp.int32)
    rev0 = edge_indices_reverse[:, 0].astype(jnp.int32)

    p, q = _make_pq(node_attributes, W_e[:d], W_e[d:2 * d])
    g = _sc_gather_sum(p, q, idx1, idx0)
    h1, _, _, a, c = _edge_mlp_stats(g, edge_attributes, W_e[2 * d:],
                                     b_e, gamma_e, beta_e)
    eu, aggp = _sc_affine_scatter(h1, rev0, a.reshape(d), c.reshape(d), n)
    aggp = aggp[:, :n]
    node_final = _node_mlp(node_attributes, aggp,
                           W_g1[:d], W_g1[d:], b_g1, gamma_g1, beta_g1,
                           W_g2, b_g2, gamma_g2, beta_g2,
                           gamma_n, beta_n, W_nu, b_nu)
    return (node_final, eu)
